# Initial kernel scaffold; baseline (speedup 1.0000x reference)
import jax
import jax.numpy as jnp
from jax.experimental import pallas as pl


def kernel(te, tod_table, dow_table):
    raise NotImplementedError("write your pallas kernel here")


# speedup vs baseline: 6.0175x; 6.0175x over previous
import functools

import jax
import jax.numpy as jnp
from jax import lax
from jax.experimental import pallas as pl
from jax.experimental.pallas import tpu as pltpu
from jax.experimental.pallas import tpu_sc as plsc

STEPS_PER_DAY = 288
DOW_ROWS = 7
TE_DIM = 64
B, T = 4096, 200
ROWS = B * T

NUM_CORES = 2
NUM_SUBCORES = 16
NW = NUM_CORES * NUM_SUBCORES
PER_W = ROWS // NW
CHUNK = 512
IDX_SLICE = 128
N_CHUNK = PER_W // CHUNK
LANES = 16


def _fused_table_body(tod_ref, dow_ref, out_ref):
    tod = tod_ref[...].reshape(STEPS_PER_DAY, 1, TE_DIM)
    dow = dow_ref[...].reshape(1, DOW_ROWS, TE_DIM)
    out_ref[...] = tod + dow


def _build_fused_table(tod_table, dow_table):
    f3 = pl.pallas_call(
        _fused_table_body,
        out_shape=jax.ShapeDtypeStruct((STEPS_PER_DAY, DOW_ROWS, TE_DIM), jnp.float32),
    )(tod_table, dow_table)
    return f3.reshape(STEPS_PER_DAY * DOW_ROWS, TE_DIM)


_MESH = plsc.VectorSubcoreMesh(core_axis_name="c", subcore_axis_name="s")


@functools.partial(
    pl.kernel,
    mesh=_MESH,
    compiler_params=pltpu.CompilerParams(use_tc_tiling_on_sc=False),
    out_type=jax.ShapeDtypeStruct((ROWS, TE_DIM), jnp.float32),
    scratch_types=[
        pltpu.VMEM((CHUNK,), jnp.int32),
        pltpu.VMEM((CHUNK,), jnp.int32),
        pltpu.VMEM((CHUNK,), jnp.int32),
        pltpu.VMEM((CHUNK, TE_DIM), jnp.float32),
        pltpu.SemaphoreType.DMA,
    ],
)
def _sc_gather(tod_hbm, dow_hbm, fused_hbm, out_hbm, tod_v, dow_v, idx_v, rows_v, sem):
    wid = lax.axis_index("s") * NUM_CORES + lax.axis_index("c")
    base_w = wid * PER_W

    def chunk_body(ci, carry):
        base = base_w + ci * CHUNK
        pltpu.sync_copy(tod_hbm.at[pl.ds(base, CHUNK)], tod_v)
        pltpu.sync_copy(dow_hbm.at[pl.ds(base, CHUNK)], dow_v)
        for i in range(CHUNK // LANES):
            sl = pl.ds(i * LANES, LANES)
            idx_v[sl] = tod_v[sl] * DOW_ROWS + jnp.minimum(dow_v[sl], DOW_ROWS - 1)
        copies = []
        for j in range(CHUNK // IDX_SLICE):
            copies.append(
                pltpu.async_copy(
                    fused_hbm.at[idx_v.at[pl.ds(j * IDX_SLICE, IDX_SLICE)]],
                    rows_v.at[pl.ds(j * IDX_SLICE, IDX_SLICE), :],
                    sem,
                )
            )
        for c in copies:
            c.wait()
        pltpu.sync_copy(rows_v, out_hbm.at[pl.ds(base, CHUNK), :])
        return carry

    lax.fori_loop(0, N_CHUNK, chunk_body, 0)


def kernel(te, tod_table, dow_table):
    fused = _build_fused_table(tod_table, dow_table)
    tod_ids = te[..., 0].reshape(ROWS)
    dow_ids = te[..., 1].reshape(ROWS)
    out = _sc_gather(tod_ids, dow_ids, fused)
    return out.reshape(B, T, TE_DIM)
